# Initial kernel scaffold; baseline (speedup 1.0000x reference)
import jax
import jax.numpy as jnp
from jax.experimental import pallas as pl


def kernel(nodes, neigh_idx, features, weight):
    raise NotImplementedError("write your pallas kernel here")


# speedup vs baseline: 1.1720x; 1.1720x over previous
import functools

import jax
import jax.numpy as jnp
from jax import lax
from jax.experimental import pallas as pl
from jax.experimental.pallas import tpu as pltpu
from jax.experimental.pallas import tpu_sc as plsc

D = 256
S = 16
EMB = 256
NC = 2
NS = 16
NW = NC * NS
SEEDS_PER_W = 320
BP = NW * SEEDS_PER_W
CS = 8
NCHUNK = SEEDS_PER_W // CS
TB = 2048


def _make_sc_gather_sum():
    mesh = plsc.VectorSubcoreMesh(core_axis_name="c", subcore_axis_name="s")

    @functools.partial(
        pl.kernel,
        mesh=mesh,
        out_type=(
            jax.ShapeDtypeStruct((BP, D), jnp.float32),
            jax.ShapeDtypeStruct((BP, D), jnp.float32),
        ),
        scratch_types=[
            pltpu.VMEM((SEEDS_PER_W * S,), jnp.int32),
            pltpu.VMEM((SEEDS_PER_W,), jnp.int32),
            pltpu.VMEM((CS * S, D), jnp.float32),
            pltpu.VMEM((CS, D), jnp.float32),
            pltpu.VMEM((CS, D), jnp.float32),
            pltpu.SemaphoreType.DMA,
        ],
    )
    def sc_gather_sum(feat_hbm, nodes_hbm, neigh_hbm, self_out, sum_out,
                      nidx_v, sidx_v, nbuf, sbuf, acc, gsem):
        wid = lax.axis_index("s") * NC + lax.axis_index("c")
        base = pl.multiple_of(wid * SEEDS_PER_W, SEEDS_PER_W)
        pltpu.sync_copy(neigh_hbm.at[pl.ds(base * S, SEEDS_PER_W * S)], nidx_v)
        pltpu.sync_copy(nodes_hbm.at[pl.ds(base, SEEDS_PER_W)], sidx_v)

        def chunk_body(g, _):
            off_n = pl.multiple_of(g * (CS * S), CS * S)
            off_s = pl.multiple_of(g * CS, CS)
            cp_n = pltpu.async_copy(
                feat_hbm.at[nidx_v.at[pl.ds(off_n, CS * S)]], nbuf, gsem)
            cp_s = pltpu.async_copy(
                feat_hbm.at[sidx_v.at[pl.ds(off_s, CS)]], sbuf, gsem)
            cp_n.wait()
            cp_s.wait()

            def seed_body(s0, _):
                r0 = s0 * S
                for v in range(D // 16):
                    a = nbuf[r0, pl.ds(v * 16, 16)]
                    for r in range(1, S):
                        a = a + nbuf[r0 + r, pl.ds(v * 16, 16)]
                    acc[s0, pl.ds(v * 16, 16)] = a
                return 0

            lax.fori_loop(0, CS, seed_body, 0, unroll=False)

            out_off = pl.multiple_of(base + g * CS, CS)
            pltpu.sync_copy(sbuf, self_out.at[pl.ds(out_off, CS)])
            pltpu.sync_copy(acc, sum_out.at[pl.ds(out_off, CS)])
            return 0

        lax.fori_loop(0, NCHUNK, chunk_body, 0, unroll=False)

    return sc_gather_sum


_sc_gather_sum = _make_sc_gather_sum()


def _mm_body(w_ref, s_ref, m_ref, o_ref):
    w = w_ref[...]
    s = s_ref[...]
    m = m_ref[...] * (1.0 / S)
    a = lax.dot_general(w[:, :D], s, (((1,), (1,)), ((), ())),
                        preferred_element_type=jnp.float32)
    b = lax.dot_general(w[:, D:], m, (((1,), (1,)), ((), ())),
                        preferred_element_type=jnp.float32)
    o_ref[...] = jnp.maximum(a + b, 0.0)


def kernel(nodes, neigh_idx, features, weight):
    batch = nodes.shape[0]
    pad = BP - batch
    nodes_p = jnp.concatenate(
        [nodes.astype(jnp.int32), jnp.zeros((pad,), jnp.int32)])
    neigh_p = jnp.concatenate(
        [neigh_idx.astype(jnp.int32).reshape(-1),
         jnp.zeros((pad * S,), jnp.int32)])

    selfs, sums = _sc_gather_sum(features, nodes_p, neigh_p)

    out_full = pl.pallas_call(
        _mm_body,
        grid=(BP // TB,),
        in_specs=[
            pl.BlockSpec((EMB, 2 * D), lambda i: (0, 0)),
            pl.BlockSpec((TB, D), lambda i: (i, 0)),
            pl.BlockSpec((TB, D), lambda i: (i, 0)),
        ],
        out_specs=pl.BlockSpec((EMB, TB), lambda i: (0, i)),
        out_shape=jax.ShapeDtypeStruct((EMB, BP), jnp.float32),
    )(weight, selfs, sums)
    return out_full[:, :batch]


# speedup vs baseline: 1.5034x; 1.2827x over previous
import functools

import jax
import jax.numpy as jnp
from jax import lax
from jax.experimental import pallas as pl
from jax.experimental.pallas import tpu as pltpu
from jax.experimental.pallas import tpu_sc as plsc

D = 256
S = 16
EMB = 256
NC = 2
NS = 16
NW = NC * NS
SEEDS_PER_W = 320
BP = NW * SEEDS_PER_W
CS = 8
NCHUNK = SEEDS_PER_W // CS
TB = 2048


def _make_sc_gather_sum():
    mesh = plsc.VectorSubcoreMesh(core_axis_name="c", subcore_axis_name="s")

    @functools.partial(
        pl.kernel,
        mesh=mesh,
        out_type=jax.ShapeDtypeStruct((BP, 2 * D), jnp.float32),
        scratch_types=[
            pltpu.VMEM((SEEDS_PER_W * S,), jnp.int32),
            pltpu.VMEM((SEEDS_PER_W,), jnp.int32),
            pltpu.VMEM((CS * S, D), jnp.float32),
            pltpu.VMEM((CS * S, D), jnp.float32),
            pltpu.VMEM((CS, D), jnp.float32),
            pltpu.VMEM((CS, D), jnp.float32),
            pltpu.VMEM((CS, 2 * D), jnp.float32),
            pltpu.VMEM((CS, 2 * D), jnp.float32),
            pltpu.SemaphoreType.DMA,
            pltpu.SemaphoreType.DMA,
            pltpu.SemaphoreType.DMA,
            pltpu.SemaphoreType.DMA,
        ],
    )
    def sc_gather_sum(feat_hbm, nodes_hbm, neigh_hbm, comb_out,
                      nidx_v, sidx_v, nbuf0, nbuf1, sbuf0, sbuf1,
                      obuf0, obuf1, gsem0, gsem1, osem0, osem1):
        wid = lax.axis_index("s") * NC + lax.axis_index("c")
        base = pl.multiple_of(wid * SEEDS_PER_W, SEEDS_PER_W)
        pltpu.sync_copy(neigh_hbm.at[pl.ds(base * S, SEEDS_PER_W * S)], nidx_v)
        pltpu.sync_copy(nodes_hbm.at[pl.ds(base, SEEDS_PER_W)], sidx_v)

        nbufs = (nbuf0, nbuf1)
        sbufs = (sbuf0, sbuf1)
        obufs = (obuf0, obuf1)
        gsems = (gsem0, gsem1)
        osems = (osem0, osem1)

        def fire_gather(g, b):
            off_n = pl.multiple_of(g * (CS * S), CS * S)
            off_s = pl.multiple_of(g * CS, CS)
            pltpu.async_copy(
                feat_hbm.at[nidx_v.at[pl.ds(off_n, CS * S)]], nbufs[b], gsems[b])
            pltpu.async_copy(
                feat_hbm.at[sidx_v.at[pl.ds(off_s, CS)]], sbufs[b], gsems[b])

        def wait_gather(b):
            pltpu.make_async_copy(
                feat_hbm.at[pl.ds(0, CS * S)], nbufs[b], gsems[b]).wait()
            pltpu.make_async_copy(
                feat_hbm.at[pl.ds(0, CS)], sbufs[b], gsems[b]).wait()

        def fire_out(g, b):
            row = pl.multiple_of(base + g * CS, CS)
            pltpu.async_copy(obufs[b], comb_out.at[pl.ds(row, CS)], osems[b])

        def drain_out(b):
            pltpu.make_async_copy(
                obufs[b], comb_out.at[pl.ds(0, CS)], osems[b]).wait()

        def compute(b):
            nb, sb, ob = nbufs[b], sbufs[b], obufs[b]

            def seed_body(s0, _):
                r0 = s0 * S
                for v in range(D // 16):
                    a = nb[r0, pl.ds(v * 16, 16)]
                    for r in range(1, S):
                        a = a + nb[r0 + r, pl.ds(v * 16, 16)]
                    ob[s0, pl.ds(D + v * 16, 16)] = a
                    ob[s0, pl.ds(v * 16, 16)] = sb[s0, pl.ds(v * 16, 16)]
                return 0

            lax.fori_loop(0, CS, seed_body, 0, unroll=False)

        fire_gather(0, 0)
        fire_gather(1, 1)

        def pair_body(p, _):
            for b in range(2):
                g = p * 2 + b
                wait_gather(b)
                compute(b)

                @pl.when(p > 0)
                def _():
                    drain_out(b)

                fire_out(g, b)

                @pl.when(g + 2 < NCHUNK)
                def _():
                    fire_gather(g + 2, b)
            return 0

        lax.fori_loop(0, NCHUNK // 2, pair_body, 0, unroll=False)
        drain_out(0)
        drain_out(1)

    return sc_gather_sum


_sc_gather_sum = _make_sc_gather_sum()


def _mm_body(w_ref, c_ref, o_ref):
    w = w_ref[...]
    s = c_ref[:, :D]
    m = c_ref[:, D:] * (1.0 / S)
    a = lax.dot_general(w[:, :D], s, (((1,), (1,)), ((), ())),
                        preferred_element_type=jnp.float32)
    b = lax.dot_general(w[:, D:], m, (((1,), (1,)), ((), ())),
                        preferred_element_type=jnp.float32)
    o_ref[...] = jnp.maximum(a + b, 0.0)


def kernel(nodes, neigh_idx, features, weight):
    batch = nodes.shape[0]
    pad = BP - batch
    nodes_p = jnp.concatenate(
        [nodes.astype(jnp.int32), jnp.zeros((pad,), jnp.int32)])
    neigh_p = jnp.concatenate(
        [neigh_idx.astype(jnp.int32).reshape(-1),
         jnp.zeros((pad * S,), jnp.int32)])

    combined = _sc_gather_sum(features, nodes_p, neigh_p)

    out_full = pl.pallas_call(
        _mm_body,
        grid=(BP // TB,),
        in_specs=[
            pl.BlockSpec((EMB, 2 * D), lambda i: (0, 0)),
            pl.BlockSpec((TB, 2 * D), lambda i: (i, 0)),
        ],
        out_specs=pl.BlockSpec((EMB, TB), lambda i: (0, i)),
        out_shape=jax.ShapeDtypeStruct((EMB, BP), jnp.float32),
    )(weight, combined)
    return out_full[:, :batch]
